# Initial kernel scaffold; baseline (speedup 1.0000x reference)
import jax
import jax.numpy as jnp
from jax.experimental import pallas as pl


def kernel(x, W):
    raise NotImplementedError("write your pallas kernel here")


# speedup vs baseline: 1.9454x; 1.9454x over previous
import functools

import jax
import jax.numpy as jnp
from jax.experimental import pallas as pl


def _vq_body(x_ref, w_ref, wt_ref, wn_ref, q_ref, i_ref, loss_ref, *, K):
    b = pl.program_id(0)
    j = pl.program_id(1)

    xb = x_ref[0]
    xsq = jnp.sum(xb * xb, axis=0, keepdims=True)
    s = jax.lax.dot_general(
        w_ref[...], xb,
        dimension_numbers=(((1,), (0,)), ((), ())),
        preferred_element_type=jnp.float32,
    )
    d = xsq + wn_ref[...] - 2.0 * s

    m = jnp.min(d, axis=0, keepdims=True)
    io = jax.lax.broadcasted_iota(jnp.int32, d.shape, 0)
    idx = jnp.min(jnp.where(d == m, io, K), axis=0, keepdims=True)
    i_ref[0] = idx

    oneh = (io == idx).astype(jnp.float32)
    q = jax.lax.dot_general(
        wt_ref[...], oneh,
        dimension_numbers=(((1,), (0,)), ((), ())),
        precision=jax.lax.Precision.HIGHEST,
        preferred_element_type=jnp.float32,
    )

    delta = q - xb
    q_ref[0] = xb + delta

    part = jnp.sum(delta * delta).reshape(1, 1)

    @pl.when((b == 0) & (j == 0))
    def _init():
        loss_ref[...] = part

    @pl.when((b > 0) | (j > 0))
    def _acc():
        loss_ref[...] = loss_ref[...] + part


def kernel(x, W):
    B, C, H, Wd = x.shape
    K, Ck = W.shape
    HW = H * Wd
    CHUNK = 2048
    NJ = HW // CHUNK

    xr = x.reshape(B, C, HW)
    wn = jnp.sum(W ** 2, axis=1).reshape(K, 1)
    Wt = W.T

    q, idx, loss = pl.pallas_call(
        functools.partial(_vq_body, K=K),
        grid=(B, NJ),
        in_specs=[
            pl.BlockSpec((1, C, CHUNK), lambda b, j: (b, 0, j)),
            pl.BlockSpec((K, Ck), lambda b, j: (0, 0)),
            pl.BlockSpec((Ck, K), lambda b, j: (0, 0)),
            pl.BlockSpec((K, 1), lambda b, j: (0, 0)),
        ],
        out_specs=[
            pl.BlockSpec((1, C, CHUNK), lambda b, j: (b, 0, j)),
            pl.BlockSpec((1, 1, CHUNK), lambda b, j: (b, 0, j)),
            pl.BlockSpec((1, 1), lambda b, j: (0, 0)),
        ],
        out_shape=[
            jax.ShapeDtypeStruct((B, C, HW), jnp.float32),
            jax.ShapeDtypeStruct((B, 1, HW), jnp.int32),
            jax.ShapeDtypeStruct((1, 1), jnp.float32),
        ],
    )(xr, W, Wt, wn)

    n = float(B * C * H * Wd)
    mloss = loss[0, 0] / n
    vq_loss = mloss + 0.25 * mloss
    return (q.reshape(B, C, H, Wd), vq_loss, idx.reshape(B, H, Wd))


# speedup vs baseline: 3.0311x; 1.5581x over previous
import functools

import jax
import jax.numpy as jnp
from jax.experimental import pallas as pl


def _vq_body(x_ref, w_ref, wt_ref, wn_ref, q_ref, i_ref, loss_ref, *, K):
    b = pl.program_id(0)
    j = pl.program_id(1)

    xb = x_ref[0]
    xsq = jnp.sum(xb * xb, axis=0, keepdims=True)
    s = jax.lax.dot_general(
        w_ref[...], xb,
        dimension_numbers=(((1,), (0,)), ((), ())),
        preferred_element_type=jnp.float32,
    )
    d = xsq + wn_ref[...] - 2.0 * s

    m = jnp.min(d, axis=0, keepdims=True)
    io = jax.lax.broadcasted_iota(jnp.int32, d.shape, 0)
    idx = jnp.min(jnp.where(d == m, io, K), axis=0, keepdims=True)
    i_ref[0] = idx

    oneh = (io == idx).astype(jnp.bfloat16)
    q = jax.lax.dot_general(
        wt_ref[...], oneh,
        dimension_numbers=(((1,), (0,)), ((), ())),
        preferred_element_type=jnp.float32,
    )

    delta = q - xb
    q_ref[0] = xb + delta

    part = jnp.sum(delta * delta).reshape(1, 1)

    @pl.when((b == 0) & (j == 0))
    def _init():
        loss_ref[...] = part

    @pl.when((b > 0) | (j > 0))
    def _acc():
        loss_ref[...] = loss_ref[...] + part


def kernel(x, W):
    B, C, H, Wd = x.shape
    K, Ck = W.shape
    HW = H * Wd
    CHUNK = 2048
    NJ = HW // CHUNK

    xr = x.reshape(B, C, HW)
    wn = jnp.sum(W ** 2, axis=1).reshape(K, 1)
    Wt = W.T.astype(jnp.bfloat16)

    q, idx, loss = pl.pallas_call(
        functools.partial(_vq_body, K=K),
        grid=(B, NJ),
        in_specs=[
            pl.BlockSpec((1, C, CHUNK), lambda b, j: (b, 0, j)),
            pl.BlockSpec((K, Ck), lambda b, j: (0, 0)),
            pl.BlockSpec((Ck, K), lambda b, j: (0, 0)),
            pl.BlockSpec((K, 1), lambda b, j: (0, 0)),
        ],
        out_specs=[
            pl.BlockSpec((1, C, CHUNK), lambda b, j: (b, 0, j)),
            pl.BlockSpec((1, 1, CHUNK), lambda b, j: (b, 0, j)),
            pl.BlockSpec((1, 1), lambda b, j: (0, 0)),
        ],
        out_shape=[
            jax.ShapeDtypeStruct((B, C, HW), jnp.float32),
            jax.ShapeDtypeStruct((B, 1, HW), jnp.int32),
            jax.ShapeDtypeStruct((1, 1), jnp.float32),
        ],
    )(xr, W, Wt, wn)

    n = float(B * C * H * Wd)
    mloss = loss[0, 0] / n
    vq_loss = mloss + 0.25 * mloss
    return (q.reshape(B, C, H, Wd), vq_loss, idx.reshape(B, H, Wd))


# speedup vs baseline: 3.0318x; 1.0002x over previous
import functools

import jax
import jax.numpy as jnp
from jax.experimental import pallas as pl
from jax.experimental.pallas import tpu as pltpu


def _vq_body(x_ref, w_ref, wt_ref, wn_ref, q_ref, i_ref, loss_ref, *, K):
    b = pl.program_id(0)
    j = pl.program_id(1)

    xb = x_ref[0]
    xsq = jnp.sum(xb * xb, axis=0, keepdims=True)
    s = jax.lax.dot_general(
        w_ref[...], xb,
        dimension_numbers=(((1,), (0,)), ((), ())),
        preferred_element_type=jnp.float32,
    )
    d = xsq + wn_ref[...] - 2.0 * s

    m = jnp.min(d, axis=0, keepdims=True)
    io = jax.lax.broadcasted_iota(jnp.int32, d.shape, 0)
    idx = jnp.min(jnp.where(d == m, io, K), axis=0, keepdims=True)
    i_ref[0] = idx

    oneh = (io == idx).astype(jnp.bfloat16)
    q = jax.lax.dot_general(
        wt_ref[...], oneh,
        dimension_numbers=(((1,), (0,)), ((), ())),
        preferred_element_type=jnp.float32,
    )

    delta = q - xb
    q_ref[0] = xb + delta

    part = jnp.sum(delta * delta).reshape(1, 1, 1)

    @pl.when(j == 0)
    def _init():
        loss_ref[...] = part

    @pl.when(j > 0)
    def _acc():
        loss_ref[...] = loss_ref[...] + part


def kernel(x, W):
    B, C, H, Wd = x.shape
    K, Ck = W.shape
    HW = H * Wd
    CHUNK = 2048
    NJ = HW // CHUNK

    xr = x.reshape(B, C, HW)
    wn = jnp.sum(W ** 2, axis=1).reshape(K, 1)
    Wt = W.T.astype(jnp.bfloat16)

    q, idx, loss = pl.pallas_call(
        functools.partial(_vq_body, K=K),
        grid=(B, NJ),
        in_specs=[
            pl.BlockSpec((1, C, CHUNK), lambda b, j: (b, 0, j)),
            pl.BlockSpec((K, Ck), lambda b, j: (0, 0)),
            pl.BlockSpec((Ck, K), lambda b, j: (0, 0)),
            pl.BlockSpec((K, 1), lambda b, j: (0, 0)),
        ],
        out_specs=[
            pl.BlockSpec((1, C, CHUNK), lambda b, j: (b, 0, j)),
            pl.BlockSpec((1, 1, CHUNK), lambda b, j: (b, 0, j)),
            pl.BlockSpec((1, 1, 1), lambda b, j: (b, 0, 0)),
        ],
        out_shape=[
            jax.ShapeDtypeStruct((B, C, HW), jnp.float32),
            jax.ShapeDtypeStruct((B, 1, HW), jnp.int32),
            jax.ShapeDtypeStruct((B, 1, 1), jnp.float32),
        ],
        compiler_params=pltpu.CompilerParams(
            dimension_semantics=("parallel", "arbitrary"),
        ),
    )(xr, W, Wt, wn)

    n = float(B * C * H * Wd)
    mloss = jnp.sum(loss) / n
    vq_loss = mloss + 0.25 * mloss
    return (q.reshape(B, C, H, Wd), vq_loss, idx.reshape(B, H, Wd))


# speedup vs baseline: 4.7397x; 1.5633x over previous
import functools

import jax
import jax.numpy as jnp
from jax.experimental import pallas as pl
from jax.experimental.pallas import tpu as pltpu


def _vq_body(x_ref, w_ref, wt_ref, wn_ref, q_ref, i_ref, loss_ref, *, K):
    j = pl.program_id(1)

    C, BH, Wd = x_ref.shape[1:]
    n = BH * Wd
    xb = x_ref[0].reshape(C, n)
    xsq = jnp.sum(xb * xb, axis=0, keepdims=True)
    s = jax.lax.dot_general(
        w_ref[...], xb,
        dimension_numbers=(((1,), (0,)), ((), ())),
        preferred_element_type=jnp.float32,
    )
    d = xsq + wn_ref[...] - 2.0 * s

    m = jnp.min(d, axis=0, keepdims=True)
    io = jax.lax.broadcasted_iota(jnp.int32, d.shape, 0)
    idx = jnp.min(jnp.where(d == m, io, K), axis=0, keepdims=True)
    i_ref[0] = idx

    oneh = (io == idx).astype(jnp.bfloat16)
    q = jax.lax.dot_general(
        wt_ref[...], oneh,
        dimension_numbers=(((1,), (0,)), ((), ())),
        preferred_element_type=jnp.float32,
    )

    delta = q - xb
    q_ref[0] = (xb + delta).reshape(C, BH, Wd)

    part = jnp.sum(delta * delta).reshape(1, 1, 1)

    @pl.when(j == 0)
    def _init():
        loss_ref[...] = part

    @pl.when(j > 0)
    def _acc():
        loss_ref[...] = loss_ref[...] + part


def kernel(x, W):
    B, C, H, Wd = x.shape
    K, Ck = W.shape
    BH = 32
    NJ = H // BH

    wn = jnp.sum(W ** 2, axis=1).reshape(K, 1)
    Wt = W.T.astype(jnp.bfloat16)

    q, idx, loss = pl.pallas_call(
        functools.partial(_vq_body, K=K),
        grid=(B, NJ),
        in_specs=[
            pl.BlockSpec((1, C, BH, Wd), lambda b, j: (b, 0, j, 0)),
            pl.BlockSpec((K, Ck), lambda b, j: (0, 0)),
            pl.BlockSpec((Ck, K), lambda b, j: (0, 0)),
            pl.BlockSpec((K, 1), lambda b, j: (0, 0)),
        ],
        out_specs=[
            pl.BlockSpec((1, C, BH, Wd), lambda b, j: (b, 0, j, 0)),
            pl.BlockSpec((1, 1, BH * Wd), lambda b, j: (b, 0, j)),
            pl.BlockSpec((1, 1, 1), lambda b, j: (b, 0, 0)),
        ],
        out_shape=[
            jax.ShapeDtypeStruct((B, C, H, Wd), jnp.float32),
            jax.ShapeDtypeStruct((B, 1, H * Wd), jnp.int32),
            jax.ShapeDtypeStruct((B, 1, 1), jnp.float32),
        ],
        compiler_params=pltpu.CompilerParams(
            dimension_semantics=("parallel", "arbitrary"),
        ),
    )(x, W, Wt, wn)

    n = float(B * C * H * Wd)
    mloss = jnp.sum(loss) / n
    vq_loss = mloss + 0.25 * mloss
    return (q, vq_loss, idx.reshape(B, H, Wd))


# speedup vs baseline: 6.1197x; 1.2912x over previous
import functools

import jax
import jax.numpy as jnp
from jax.experimental import pallas as pl
from jax.experimental.pallas import tpu as pltpu


def _vq_body(x_ref, w_ref, wt_ref, wn_ref, q_ref, i_ref, loss_ref, *, K):
    j = pl.program_id(1)

    C, BH, Wd = x_ref.shape[1:]
    n = BH * Wd
    xb = x_ref[0].reshape(C, n)
    xsq = jnp.sum(xb * xb, axis=0, keepdims=True)
    s2 = jax.lax.dot_general(
        w_ref[...], xb,
        dimension_numbers=(((1,), (0,)), ((), ())),
        preferred_element_type=jnp.float32,
    )
    d = xsq + wn_ref[...] - s2

    idx = jnp.argmin(d, axis=0).reshape(1, n)
    i_ref[0] = idx
    io = jax.lax.broadcasted_iota(jnp.int32, d.shape, 0)

    oneh = (io == idx).astype(jnp.bfloat16)
    q = jax.lax.dot_general(
        wt_ref[...], oneh,
        dimension_numbers=(((1,), (0,)), ((), ())),
        preferred_element_type=jnp.float32,
    )

    delta = q - xb
    q_ref[0] = (xb + delta).reshape(C, BH, Wd)

    part = jnp.sum(delta * delta).reshape(1, 1, 1)

    @pl.when(j == 0)
    def _init():
        loss_ref[...] = part

    @pl.when(j > 0)
    def _acc():
        loss_ref[...] = loss_ref[...] + part


def kernel(x, W):
    B, C, H, Wd = x.shape
    K, Ck = W.shape
    BH = 32
    NJ = H // BH

    wn = jnp.sum(W ** 2, axis=1).reshape(K, 1)
    Wt = W.T.astype(jnp.bfloat16)
    W2 = W + W

    q, idx, loss = pl.pallas_call(
        functools.partial(_vq_body, K=K),
        grid=(B, NJ),
        in_specs=[
            pl.BlockSpec((1, C, BH, Wd), lambda b, j: (b, 0, j, 0)),
            pl.BlockSpec((K, Ck), lambda b, j: (0, 0)),
            pl.BlockSpec((Ck, K), lambda b, j: (0, 0)),
            pl.BlockSpec((K, 1), lambda b, j: (0, 0)),
        ],
        out_specs=[
            pl.BlockSpec((1, C, BH, Wd), lambda b, j: (b, 0, j, 0)),
            pl.BlockSpec((1, 1, BH * Wd), lambda b, j: (b, 0, j)),
            pl.BlockSpec((1, 1, 1), lambda b, j: (b, 0, 0)),
        ],
        out_shape=[
            jax.ShapeDtypeStruct((B, C, H, Wd), jnp.float32),
            jax.ShapeDtypeStruct((B, 1, H * Wd), jnp.int32),
            jax.ShapeDtypeStruct((B, 1, 1), jnp.float32),
        ],
        compiler_params=pltpu.CompilerParams(
            dimension_semantics=("parallel", "arbitrary"),
        ),
    )(x, W2, Wt, wn)

    n = float(B * C * H * Wd)
    mloss = jnp.sum(loss) / n
    vq_loss = mloss + 0.25 * mloss
    return (q, vq_loss, idx.reshape(B, H, Wd))
